# speedup vs baseline: 1.9924x; 1.9924x over previous
import functools

import jax
import jax.numpy as jnp
from jax import lax
from jax.experimental import pallas as pl
from jax.experimental.pallas import tpu as pltpu
from jax.experimental.pallas import tpu_sc as plsc

D = 1024
NC = 2
NS = 16
NW = NC * NS
B = 4 * 8192
B_PER_W = B // NW
CHUNK = 32
N_CHUNKS = B_PER_W // CHUNK

_mesh = plsc.VectorSubcoreMesh(core_axis_name="c", subcore_axis_name="s")


@functools.partial(
    pl.kernel,
    mesh=_mesh,
    out_type=jax.ShapeDtypeStruct((B, D), jnp.float32),
    scratch_types=[
        pltpu.VMEM((N_CHUNKS, CHUNK), jnp.int32),
        pltpu.VMEM((CHUNK, D), jnp.float32),
        pltpu.SemaphoreType.DMA,
    ],
)
def _gather_kernel(t_hbm, pe_hbm, out_hbm, idx_v, rows_v, sem):
    wid = lax.axis_index("s") * NC + lax.axis_index("c")
    base = wid * B_PER_W
    pltpu.sync_copy(t_hbm.at[pl.ds(wid * N_CHUNKS, N_CHUNKS)], idx_v)

    def body(j, carry):
        pltpu.async_copy(pe_hbm.at[idx_v.at[j]], rows_v, sem).wait()
        pltpu.sync_copy(rows_v, out_hbm.at[pl.ds(base + j * CHUNK, CHUNK)])
        return carry

    lax.fori_loop(0, N_CHUNKS, body, 0)


def kernel(t, pe):
    t2 = t.reshape(NW * N_CHUNKS, CHUNK).astype(jnp.int32)
    out = _gather_kernel(t2, pe)
    return out.reshape(t.shape[0], t.shape[1], D)


# speedup vs baseline: 2.3592x; 1.1841x over previous
import functools

import jax
import jax.numpy as jnp
from jax import lax
from jax.experimental import pallas as pl
from jax.experimental.pallas import tpu as pltpu
from jax.experimental.pallas import tpu_sc as plsc

D = 1024
NC = 2
NS = 16
NW = NC * NS
B = 4 * 8192
B_PER_W = B // NW
CHUNK = 16
N_CHUNKS = B_PER_W // CHUNK
NB = 4
LOOKAHEAD = 2

_mesh = plsc.VectorSubcoreMesh(core_axis_name="c", subcore_axis_name="s")


@functools.partial(
    pl.kernel,
    mesh=_mesh,
    out_type=jax.ShapeDtypeStruct((B, D), jnp.float32),
    scratch_types=[
        pltpu.VMEM((N_CHUNKS, CHUNK), jnp.int32),
        pltpu.VMEM((CHUNK, D), jnp.float32),
        pltpu.VMEM((CHUNK, D), jnp.float32),
        pltpu.VMEM((CHUNK, D), jnp.float32),
        pltpu.VMEM((CHUNK, D), jnp.float32),
        pltpu.SemaphoreType.DMA,
        pltpu.SemaphoreType.DMA,
        pltpu.SemaphoreType.DMA,
        pltpu.SemaphoreType.DMA,
        pltpu.SemaphoreType.DMA,
        pltpu.SemaphoreType.DMA,
        pltpu.SemaphoreType.DMA,
        pltpu.SemaphoreType.DMA,
    ],
)
def _gather_kernel(t_hbm, pe_hbm, out_hbm, idx_v, r0, r1, r2, r3,
                   g0, g1, g2, g3, s0, s1, s2, s3):
    rows = (r0, r1, r2, r3)
    gsem = (g0, g1, g2, g3)
    ssem = (s0, s1, s2, s3)

    wid = lax.axis_index("s") * NC + lax.axis_index("c")
    base = wid * B_PER_W
    pltpu.sync_copy(t_hbm.at[pl.ds(wid * N_CHUNKS, N_CHUNKS)], idx_v)

    def g_start(j, b):
        pltpu.async_copy(pe_hbm.at[idx_v.at[j]], rows[b], gsem[b])

    def g_wait(j, b):
        pltpu.make_async_copy(pe_hbm.at[idx_v.at[j]], rows[b], gsem[b]).wait()

    def out_slice(j):
        return out_hbm.at[pl.ds(base + j * CHUNK, CHUNK)]

    def s_start(j, b):
        pltpu.async_copy(rows[b], out_slice(j), ssem[b])

    def s_wait(j, b):
        pltpu.make_async_copy(rows[b], out_slice(j), ssem[b]).wait()

    g_start(0, 0)
    g_start(1, 1)
    for j in (0, 1):
        g_wait(j, j)
        s_start(j, j)
        g_start(j + LOOKAHEAD, j + LOOKAHEAD)

    def body(k, carry):
        jbase = 2 + k * NB
        for b in range(NB):
            j = jbase + b
            bj = (2 + b) % NB
            g_wait(j, bj)
            s_start(j, bj)
            s_wait(j - LOOKAHEAD, (2 + b - LOOKAHEAD) % NB)
            g_start(j + LOOKAHEAD, (2 + b + LOOKAHEAD) % NB)
        return carry

    lax.fori_loop(0, (N_CHUNKS - 4) // NB, body, 0)

    for j in (N_CHUNKS - 2, N_CHUNKS - 1):
        b = j % NB
        g_wait(j, b)
        s_start(j, b)
        s_wait(j - LOOKAHEAD, (j - LOOKAHEAD) % NB)
    for j in (N_CHUNKS - 2, N_CHUNKS - 1):
        s_wait(j, j % NB)


def kernel(t, pe):
    t2 = t.reshape(NW * N_CHUNKS, CHUNK).astype(jnp.int32)
    out = _gather_kernel(t2, pe)
    return out.reshape(t.shape[0], t.shape[1], D)


# speedup vs baseline: 2.3806x; 1.0091x over previous
import functools

import jax
import jax.numpy as jnp
from jax import lax
from jax.experimental import pallas as pl
from jax.experimental.pallas import tpu as pltpu
from jax.experimental.pallas import tpu_sc as plsc

D = 1024
NC = 2
NS = 16
NW = NC * NS
B = 4 * 8192
B_PER_W = B // NW
CHUNK = 16
N_CHUNKS = B_PER_W // CHUNK
NB = 6
LA = 3

_mesh = plsc.VectorSubcoreMesh(core_axis_name="c", subcore_axis_name="s")


@functools.partial(
    pl.kernel,
    mesh=_mesh,
    out_type=jax.ShapeDtypeStruct((B, D), jnp.float32),
    scratch_types=(
        [pltpu.VMEM((N_CHUNKS, CHUNK), jnp.int32)]
        + [pltpu.VMEM((CHUNK, D), jnp.float32) for _ in range(NB)]
        + [pltpu.SemaphoreType.DMA for _ in range(2 * NB)]
    ),
)
def _gather_kernel(t_hbm, pe_hbm, out_hbm, idx_v, *bufs):
    rows = bufs[:NB]
    gsem = bufs[NB:2 * NB]
    ssem = bufs[2 * NB:]

    wid = lax.axis_index("s") * NC + lax.axis_index("c")
    base = wid * B_PER_W
    pltpu.sync_copy(t_hbm.at[pl.ds(wid * N_CHUNKS, N_CHUNKS)], idx_v)

    def g_start(j, b):
        pltpu.async_copy(pe_hbm.at[idx_v.at[j]], rows[b], gsem[b])

    def g_wait(j, b):
        pltpu.make_async_copy(pe_hbm.at[idx_v.at[j]], rows[b], gsem[b]).wait()

    def out_slice(j):
        return out_hbm.at[pl.ds(base + j * CHUNK, CHUNK)]

    def s_start(j, b):
        pltpu.async_copy(rows[b], out_slice(j), ssem[b])

    def s_wait(j, b):
        pltpu.make_async_copy(rows[b], out_slice(j), ssem[b]).wait()

    def step(j, jb, full_wait=True, start_next=True):
        g_wait(j, jb)
        s_start(j, jb)
        if full_wait:
            s_wait(j - LA, (jb - LA) % NB)
        if start_next:
            g_start(j + LA, (jb + LA) % NB)

    for j in range(LA):
        g_start(j, j % NB)
    for j in range(LA):
        step(j, j % NB, full_wait=False)

    K = (N_CHUNKS - 2 * LA) // NB

    def body(k, carry):
        jbase = LA + k * NB
        for b in range(NB):
            step(jbase + b, (LA + b) % NB)
        return carry

    lax.fori_loop(0, K, body, 0)

    for j in range(LA + K * NB, N_CHUNKS):
        step(j, j % NB, start_next=(j + LA < N_CHUNKS))
    for j in range(N_CHUNKS - LA, N_CHUNKS):
        s_wait(j, j % NB)


def kernel(t, pe):
    t2 = t.reshape(NW * N_CHUNKS, CHUNK).astype(jnp.int32)
    out = _gather_kernel(t2, pe)
    return out.reshape(t.shape[0], t.shape[1], D)
